# baseline (speedup 1.0000x reference)
import functools

import jax
import jax.numpy as jnp
from jax import lax
from jax.experimental import pallas as pl
from jax.experimental.pallas import tpu as pltpu
from jax.experimental.pallas import tpu_sc as plsc

B, P, D = 32, 576, 768
NW = 32
R1 = 16
R2 = 8
NCOL = D // 16
NB = 4
NS = B // 2
NG = NS // NB

_mesh = plsc.VectorSubcoreMesh(core_axis_name="c", subcore_axis_name="s")


@functools.partial(
    pl.kernel,
    mesh=_mesh,
    out_type=jax.ShapeDtypeStruct((B, P, D), jnp.float32),
    scratch_types=(
        [pltpu.VMEM((R1, D), jnp.float32)]
        + [pltpu.VMEM((R2, D), jnp.float32)]
        + [pltpu.VMEM((2, R1, D), jnp.float32)] * NB
        + [pltpu.VMEM((2, R2, D), jnp.float32)]
        + [pltpu.SemaphoreType.DMA] * (2 * NB + 2)
    ),
)
def _sc_add(x_hbm, emb_hbm, out_hbm, emb1, emb2, *rest):
    bufs1 = rest[:NB]
    buf2 = rest[NB]
    sems = rest[NB + 1:]
    l1 = sems[:NB]
    s1 = sems[NB:2 * NB]
    l2 = sems[2 * NB]
    s2 = sems[2 * NB + 1]

    wid = lax.axis_index("s") * 2 + lax.axis_index("c")
    rb1 = wid * R1
    rb2 = NW * R1 + (wid // 4) * R2
    tb0 = (wid % 4) * 8

    pltpu.sync_copy(emb_hbm.at[pl.ds(rb1, R1), :], emb1)
    pltpu.sync_copy(emb_hbm.at[pl.ds(rb2, R2), :], emb2)

    def load1(t, j):
        pltpu.async_copy(
            x_hbm.at[pl.ds(2 * t, 2), pl.ds(rb1, R1), :], bufs1[j], l1[j])

    def load2(u):
        pltpu.async_copy(
            x_hbm.at[pl.ds(tb0 + 2 * u, 2), pl.ds(rb2, R2), :], buf2, l2)

    def add_emb(buf, emb_v, nrows):
        def body(r, _):
            for c in range(NCOL):
                s = pl.ds(c * 16, 16)
                ev = emb_v[r, s]
                plsc.addupdate(buf.at[0, r, s], ev)
                plsc.addupdate(buf.at[1, r, s], ev)
            return ()
        lax.fori_loop(0, nrows, body, ())

    load1(0, 0)
    load1(1, 1)
    load2(0)

    def group(g, _):
        for j in range(NB):
            t = g * NB + j
            jn = (j + 2) % NB

            @pl.when(t >= 2)
            def _():
                pltpu.make_async_copy(
                    bufs1[jn],
                    out_hbm.at[pl.ds(2 * (t - 2), 2), pl.ds(rb1, R1), :],
                    s1[jn]).wait()

            @pl.when(t + 2 < NS)
            def _():
                load1(t + 2, jn)

            pltpu.make_async_copy(
                x_hbm.at[pl.ds(2 * t, 2), pl.ds(rb1, R1), :], bufs1[j],
                l1[j]).wait()
            add_emb(bufs1[j], emb1, R1)
            pltpu.async_copy(
                bufs1[j], out_hbm.at[pl.ds(2 * t, 2), pl.ds(rb1, R1), :],
                s1[j])

            if j == 2:
                pltpu.make_async_copy(
                    x_hbm.at[pl.ds(tb0 + 2 * g, 2), pl.ds(rb2, R2), :],
                    buf2, l2).wait()
                add_emb(buf2, emb2, R2)
                pltpu.async_copy(
                    buf2,
                    out_hbm.at[pl.ds(tb0 + 2 * g, 2), pl.ds(rb2, R2), :], s2)
            if j == 3:
                pltpu.make_async_copy(
                    buf2,
                    out_hbm.at[pl.ds(tb0 + 2 * g, 2), pl.ds(rb2, R2), :],
                    s2).wait()

                @pl.when(g + 1 < NG)
                def _():
                    load2(g + 1)
        return ()

    lax.fori_loop(0, NG, group, ())

    for t in (NS - 2, NS - 1):
        pltpu.make_async_copy(
            bufs1[t % NB],
            out_hbm.at[pl.ds(2 * t, 2), pl.ds(rb1, R1), :], s1[t % NB]).wait()


def kernel(x, embedding):
    return _sc_add(x, embedding)


# speedup vs baseline: 1.0732x; 1.0732x over previous
import functools

import jax
import jax.numpy as jnp
from jax import lax
from jax.experimental import pallas as pl
from jax.experimental.pallas import tpu as pltpu
from jax.experimental.pallas import tpu_sc as plsc

B, P, D = 32, 576, 768
NW = 32
R1 = 16
R2 = 8
NCOL = D // 16
NB = 4
NG = B // NB

_mesh = plsc.VectorSubcoreMesh(core_axis_name="c", subcore_axis_name="s")


@functools.partial(
    pl.kernel,
    mesh=_mesh,
    out_type=jax.ShapeDtypeStruct((B, P, D), jnp.float32),
    scratch_types=(
        [pltpu.VMEM((R1, D), jnp.float32)]
        + [pltpu.VMEM((R2, D), jnp.float32)]
        + [pltpu.VMEM((R1, D), jnp.float32)] * NB
        + [pltpu.VMEM((R2, D), jnp.float32)]
        + [pltpu.SemaphoreType.DMA] * (2 * NB + 2 + 1)
    ),
)
def _sc_add(x_hbm, emb_hbm, out_hbm, emb1, emb2, *rest):
    bufs1 = rest[:NB]
    buf2 = rest[NB]
    sems = rest[NB + 1:]
    l1 = sems[:NB]
    s1 = sems[NB:2 * NB]
    l2 = sems[2 * NB]
    s2 = sems[2 * NB + 1]
    le = sems[2 * NB + 2]

    wid = lax.axis_index("s") * 2 + lax.axis_index("c")
    rb1 = wid * R1
    rb2 = NW * R1 + (wid // 4) * R2
    tb0 = (wid % 4) * 8

    def load1(b, j):
        pltpu.async_copy(x_hbm.at[b, pl.ds(rb1, R1), :], bufs1[j], l1[j])

    def load2(i):
        pltpu.async_copy(x_hbm.at[tb0 + i, pl.ds(rb2, R2), :], buf2, l2)

    def add_emb(buf, emb_v, nrows):
        def body(r, _):
            for c in range(NCOL):
                s = pl.ds(c * 16, 16)
                plsc.addupdate(buf.at[r, s], emb_v[r, s])
            return ()
        lax.fori_loop(0, nrows, body, ())

    pltpu.async_copy(emb_hbm.at[pl.ds(rb1, R1), :], emb1, le)
    load1(0, 0)
    load1(1, 1)
    load2(0)
    pltpu.make_async_copy(emb_hbm.at[pl.ds(rb1, R1), :], emb1, le).wait()
    pltpu.async_copy(emb_hbm.at[pl.ds(rb2, R2), :], emb2, le)
    pltpu.make_async_copy(emb_hbm.at[pl.ds(rb2, R2), :], emb2, le).wait()

    def group(g, _):
        for j in range(NB):
            b = g * NB + j
            jn = (j + 2) % NB

            @pl.when(b >= 2)
            def _():
                pltpu.make_async_copy(
                    bufs1[jn], out_hbm.at[b - 2, pl.ds(rb1, R1), :],
                    s1[jn]).wait()

            @pl.when(b + 2 < B)
            def _():
                load1(b + 2, jn)

            pltpu.make_async_copy(
                x_hbm.at[b, pl.ds(rb1, R1), :], bufs1[j], l1[j]).wait()
            add_emb(bufs1[j], emb1, R1)
            pltpu.async_copy(
                bufs1[j], out_hbm.at[b, pl.ds(rb1, R1), :], s1[j])

            if j == 1:
                pltpu.make_async_copy(
                    x_hbm.at[tb0 + g, pl.ds(rb2, R2), :], buf2, l2).wait()
                add_emb(buf2, emb2, R2)
                pltpu.async_copy(
                    buf2, out_hbm.at[tb0 + g, pl.ds(rb2, R2), :], s2)
            if j == 3:
                pltpu.make_async_copy(
                    buf2, out_hbm.at[tb0 + g, pl.ds(rb2, R2), :], s2).wait()

                @pl.when(g + 1 < NG)
                def _():
                    load2(g + 1)
        return ()

    lax.fori_loop(0, NG, group, ())

    for b in (B - 2, B - 1):
        pltpu.make_async_copy(
            bufs1[b % NB], out_hbm.at[b, pl.ds(rb1, R1), :], s1[b % NB]).wait()


def kernel(x, embedding):
    return _sc_add(x, embedding)
